# baseline (speedup 1.0000x reference)
import jax
import jax.numpy as jnp
from jax import lax
from jax.experimental import pallas as pl
from jax.experimental.pallas import tpu as pltpu

_NUM_CLUSTERS = 10
_EMBED = 64
_PACK = 128
_ITERS = 10
_C = 16
_C2 = 32
_CHUNKS = 4

_SHIFT = 4096.0
_BIG = 3.0e38


def _w_bias_from(cent):
    z = jnp.zeros((_C, _EMBED), jnp.float32)
    top = jnp.concatenate([-2.0 * cent, z], axis=1)
    bot = jnp.concatenate([z, -2.0 * cent], axis=1)
    w = jnp.concatenate([top, bot], axis=0).astype(jnp.bfloat16)
    cent2 = jnp.sum(cent * cent, axis=1, keepdims=True) + _SHIFT
    iota_c = lax.broadcasted_iota(jnp.int32, (_C2, 1), 0)
    bias = jnp.where((iota_c % _C) < _NUM_CLUSTERS,
                     jnp.concatenate([cent2, cent2], axis=0), _BIG)
    return w, bias


def _kmeans_body(w0_ref, bias0_ref, x_ref, cent_out_ref, zero_out_ref,
                 acc_ref, cnt_ref, w_ref, bias_ref):
    i = pl.program_id(0)
    b = pl.program_id(1)
    nb = pl.num_programs(1)
    rows = x_ref.shape[0]
    cr = rows // _CHUNKS

    @pl.when(b == 0)
    def _start_iter():
        @pl.when(i == 0)
        def _():
            w_ref[...] = w0_ref[...]
            bias_ref[...] = bias0_ref[...]

        @pl.when(i > 0)
        def _():
            accv = acc_ref[...]
            cntv = cnt_ref[...]
            sums = accv[:_C, :_EMBED] + accv[_C:, _EMBED:]
            cnt = jnp.clip(cntv[:_C] + cntv[_C:], 1.0, None)
            w, bias = _w_bias_from(sums / cnt)
            w_ref[...] = w
            bias_ref[...] = bias

        acc_ref[...] = jnp.zeros_like(acc_ref)
        cnt_ref[...] = jnp.zeros_like(cnt_ref)

    w = w_ref[...]
    bias = bias_ref[...]
    iota = lax.broadcasted_iota(jnp.int32, (_C2, 1), 0) % _C
    for c in range(_CHUNKS):
        xb = x_ref[c * cr:(c + 1) * cr, :]
        mm = lax.dot_general(w, xb, (((1,), (1,)), ((), ())),
                             preferred_element_type=jnp.float32)
        ik = lax.bitcast_convert_type(mm + bias, jnp.int32)
        key = lax.bitcast_convert_type((ik & jnp.int32(-16)) | iota,
                                       jnp.float32)
        mine = jnp.min(key[:_C], axis=0, keepdims=True)
        mino = jnp.min(key[_C:], axis=0, keepdims=True)
        minkey = jnp.concatenate([jnp.broadcast_to(mine, (_C, cr)),
                                  jnp.broadcast_to(mino, (_C, cr))], axis=0)
        sel = key == minkey
        onehot = jnp.where(sel, 1.0, 0.0)
        acc_ref[...] += lax.dot_general(
            onehot.astype(jnp.bfloat16), xb, (((1,), (0,)), ((), ())),
            preferred_element_type=jnp.float32)
        cnt_ref[...] += jnp.sum(onehot, axis=1, keepdims=True)

    @pl.when((i == _ITERS - 1) & (b == nb - 1))
    def _finish():
        accv = acc_ref[...]
        cntv = cnt_ref[...]
        sums = accv[:_C, :_EMBED] + accv[_C:, _EMBED:]
        cnt = jnp.clip(cntv[:_C] + cntv[_C:], 1.0, None)
        cent_out_ref[...] = sums / cnt
        zero_out_ref[...] = jnp.zeros_like(zero_out_ref)


def _run_kmeans(x_packed, w0, bias0, block_rows):
    n = x_packed.shape[0]
    nb = n // block_rows
    grid = (_ITERS, nb)
    cent_out, zero_out = pl.pallas_call(
        _kmeans_body,
        grid=grid,
        in_specs=[
            pl.BlockSpec((_C2, _PACK), lambda i, b: (0, 0)),
            pl.BlockSpec((_C2, 1), lambda i, b: (0, 0)),
            pl.BlockSpec((block_rows, _PACK), lambda i, b: (b, 0)),
        ],
        out_specs=[
            pl.BlockSpec((_C, _EMBED), lambda i, b: (0, 0)),
            pl.BlockSpec((1, 1), lambda i, b: (0, 0)),
        ],
        out_shape=[
            jax.ShapeDtypeStruct((_C, _EMBED), jnp.float32),
            jax.ShapeDtypeStruct((1, 1), jnp.float32),
        ],
        scratch_shapes=[
            pltpu.VMEM((_C2, _PACK), jnp.float32),
            pltpu.VMEM((_C2, 1), jnp.float32),
            pltpu.VMEM((_C2, _PACK), jnp.bfloat16),
            pltpu.VMEM((_C2, 1), jnp.float32),
        ],
        compiler_params=pltpu.CompilerParams(
            dimension_semantics=("arbitrary", "arbitrary")),
    )(w0, bias0, x_packed)
    return cent_out, zero_out


def kernel(x):
    x_packed = x.reshape(-1, _PACK).astype(jnp.bfloat16)
    ckey = jax.random.key(42)
    cents0 = jax.random.uniform(ckey, (_NUM_CLUSTERS, _EMBED), dtype=jnp.float32)
    cent = jnp.zeros((_C, _EMBED), jnp.float32).at[:_NUM_CLUSTERS].set(cents0)
    w0, bias0 = _w_bias_from(cent)
    _, zero_out = _run_kmeans(x_packed, w0, bias0, 8192)
    return zero_out.reshape(())


# speedup vs baseline: 1.2474x; 1.2474x over previous
import jax
import jax.numpy as jnp
from jax import lax
from jax.experimental import pallas as pl
from jax.experimental.pallas import tpu as pltpu

_NUM_CLUSTERS = 10
_EMBED = 64
_PACK = 128
_ITERS = 10
_C = 16
_C2 = 32
_CHUNKS = 4

_SHIFT = 4096.0
_BIG = 3.0e38


def _w_bias_from(cent):
    z = jnp.zeros((_C, _EMBED), jnp.float32)
    top = jnp.concatenate([-2.0 * cent, z], axis=1)
    bot = jnp.concatenate([z, -2.0 * cent], axis=1)
    w = jnp.concatenate([top, bot], axis=0).astype(jnp.bfloat16)
    cent2 = jnp.sum(cent * cent, axis=1, keepdims=True) + _SHIFT
    iota_c = lax.broadcasted_iota(jnp.int32, (_C2, 1), 0)
    bias = jnp.where((iota_c % _C) < _NUM_CLUSTERS,
                     jnp.concatenate([cent2, cent2], axis=0), _BIG)
    return w, bias


def _kmeans_body(w0_ref, bias0_ref, x_ref, cent_out_ref, zero_out_ref,
                 acc_ref, cnt_ref, w_ref, bias_ref):
    i = pl.program_id(0)
    b = pl.program_id(1)
    nb = pl.num_programs(1)
    rows = x_ref.shape[0]
    cr = rows // _CHUNKS

    @pl.when(b == 0)
    def _start_iter():
        @pl.when(i == 0)
        def _():
            w_ref[...] = w0_ref[...]
            bias_ref[...] = bias0_ref[...]

        @pl.when(i > 0)
        def _():
            accv = acc_ref[...]
            cntv = cnt_ref[...]
            sums = accv[:_C, :_EMBED] + accv[_C:, _EMBED:]
            cnt = jnp.clip(cntv[:_C] + cntv[_C:], 1.0, None)
            w, bias = _w_bias_from(sums / cnt)
            w_ref[...] = w
            bias_ref[...] = bias

        acc_ref[...] = jnp.zeros_like(acc_ref)
        cnt_ref[...] = jnp.zeros_like(cnt_ref)

    w = w_ref[...]
    bias = bias_ref[...]
    iota = lax.broadcasted_iota(jnp.int32, (_C2, 1), 0) % _C
    for c in range(_CHUNKS):
        xb = x_ref[c * cr:(c + 1) * cr, :]
        mm = lax.dot_general(w, xb, (((1,), (1,)), ((), ())),
                             preferred_element_type=jnp.float32)
        ik = lax.bitcast_convert_type(mm + bias, jnp.int32)
        key = lax.bitcast_convert_type((ik & jnp.int32(-16)) | iota,
                                       jnp.float32)
        mine = jnp.min(key[:_C], axis=0, keepdims=True)
        mino = jnp.min(key[_C:], axis=0, keepdims=True)
        minkey = jnp.concatenate([jnp.broadcast_to(mine, (_C, cr)),
                                  jnp.broadcast_to(mino, (_C, cr))], axis=0)
        sel = key == minkey
        onehot = jnp.where(sel, 1.0, 0.0)
        acc_ref[...] += lax.dot_general(
            onehot.astype(jnp.bfloat16), xb, (((1,), (0,)), ((), ())),
            preferred_element_type=jnp.float32)
        cnt_ref[...] += jnp.sum(onehot, axis=1, keepdims=True)

    @pl.when((i == _ITERS - 1) & (b == nb - 1))
    def _finish():
        accv = acc_ref[...]
        cntv = cnt_ref[...]
        sums = accv[:_C, :_EMBED] + accv[_C:, _EMBED:]
        cnt = jnp.clip(cntv[:_C] + cntv[_C:], 1.0, None)
        cent_out_ref[...] = sums / cnt
        zero_out_ref[...] = jnp.zeros_like(zero_out_ref)


def _run_kmeans(x_packed, w0, bias0, block_rows):
    n = x_packed.shape[0]
    nb = n // block_rows
    grid = (_ITERS, nb)
    cent_out, zero_out = pl.pallas_call(
        _kmeans_body,
        grid=grid,
        in_specs=[
            pl.BlockSpec((_C2, _PACK), lambda i, b: (0, 0)),
            pl.BlockSpec((_C2, 1), lambda i, b: (0, 0)),
            pl.BlockSpec((block_rows, _PACK), lambda i, b: (b, 0)),
        ],
        out_specs=[
            pl.BlockSpec((_C, _EMBED), lambda i, b: (0, 0)),
            pl.BlockSpec((1, 1), lambda i, b: (0, 0)),
        ],
        out_shape=[
            jax.ShapeDtypeStruct((_C, _EMBED), jnp.float32),
            jax.ShapeDtypeStruct((1, 1), jnp.float32),
        ],
        scratch_shapes=[
            pltpu.VMEM((_C2, _PACK), jnp.float32),
            pltpu.VMEM((_C2, 1), jnp.float32),
            pltpu.VMEM((_C2, _PACK), jnp.bfloat16),
            pltpu.VMEM((_C2, 1), jnp.float32),
        ],
        compiler_params=pltpu.CompilerParams(
            dimension_semantics=("arbitrary", "arbitrary")),
    )(w0, bias0, x_packed)
    return cent_out, zero_out


def kernel(x):
    x_packed = x.reshape(-1, _PACK).astype(jnp.bfloat16)
    ckey = jax.random.key(42)
    cents0 = jax.random.uniform(ckey, (_NUM_CLUSTERS, _EMBED), dtype=jnp.float32)
    cent = jnp.zeros((_C, _EMBED), jnp.float32).at[:_NUM_CLUSTERS].set(cents0)
    w0, bias0 = _w_bias_from(cent)
    _, zero_out = _run_kmeans(x_packed, w0, bias0, 16384)
    return zero_out.reshape(())
